# Initial kernel scaffold; baseline (speedup 1.0000x reference)
import jax
import jax.numpy as jnp
from jax.experimental import pallas as pl


def kernel(x, edge_index, WQ, WK, WV, WS, WE, w_a, w_g, b_g, ln1_g, ln1_b, ln2_g, ln2_b, ffn_w1, ffn_b1, ffn_w2, ffn_b2, se_w1, se_b1, se_w2, se_b2):
    raise NotImplementedError("write your pallas kernel here")


# speedup vs baseline: 6.7223x; 6.7223x over previous
import functools

import jax
import jax.numpy as jnp
from jax import lax
from jax.experimental import pallas as pl
from jax.experimental.pallas import tpu as pltpu
from jax.experimental.pallas import tpu_sc as plsc

N = 10000
E = 320000
DIM = 128
SDIM = 4
HID = 4 * DIM
R = DIM // 4

NC = 2
NS = 16
NW = NC * NS
EW = E // NW
C = 80
NCHUNK = EW // C
G = C // 16
NPAD = 10112
ROWS_PER_TILE = NPAD // NS
ZROWS = 8

_f32 = jnp.float32


def _tc1_body(x_ref, wq_ref, wk_ref, wv_ref, ws_ref, g_ref, b_ref,
              qt_ref, kt_ref, v_ref):
    xb = x_ref[...]
    m = jnp.mean(xb, axis=-1, keepdims=True)
    var = jnp.mean((xb - m) ** 2, axis=-1, keepdims=True)
    h = (xb - m) * lax.rsqrt(var + 1e-5) * g_ref[...] + b_ref[...]
    dn = (((1,), (1,)), ((), ()))
    q = lax.dot_general(h, wq_ref[...], dn, preferred_element_type=_f32)
    k = lax.dot_general(h, wk_ref[...], dn, preferred_element_type=_f32)
    v = lax.dot_general(h, wv_ref[...], dn, preferred_element_type=_f32)
    t = lax.dot_general(xb[:, DIM - 4:], ws_ref[...], dn,
                        preferred_element_type=_f32)
    qt_ref[...] = q + t
    kt_ref[...] = k - t
    v_ref[...] = v


def _tc1(x, WQ, WK, WV, WS, ln1_g, ln1_b):
    BR = 2000
    grid = N // BR
    full2 = lambda shape: pl.BlockSpec(shape, lambda i: (0, 0))
    vec = pl.BlockSpec((DIM,), lambda i: (0,))
    blk = pl.BlockSpec((BR, DIM), lambda i: (i, 0))
    return pl.pallas_call(
        _tc1_body,
        grid=(grid,),
        in_specs=[blk, full2((DIM, DIM)), full2((DIM, DIM)), full2((DIM, DIM)),
                  full2((DIM, SDIM)), vec, vec],
        out_specs=[blk, blk, blk],
        out_shape=[jax.ShapeDtypeStruct((N, DIM), _f32)] * 3,
    )(x, WQ, WK, WV, WS, ln1_g, ln1_b)


def _p1_body(dst_hbm, src_hbm, qt_hbm, kt_hbm, wa_hbm,
             ex_hbm, apart_hbm, dall_hbm,
             a_sp, dvec, svec, qbuf, abuf, exmat, denom_v,
             wav, sem1, sem2):
    c = lax.axis_index("c")
    s = lax.axis_index("s")
    wid = s * NC + c

    zero16 = jnp.zeros((16,), _f32)
    for r in range(ZROWS):
        for col in range(DIM // 16):
            abuf[r, pl.ds(col * 16, 16)] = zero16
    row0 = s * ROWS_PER_TILE
    for b in range(ROWS_PER_TILE // ZROWS):
        pltpu.sync_copy(abuf.at[pl.ds(0, ZROWS)],
                        a_sp.at[pl.ds(row0 + b * ZROWS, ZROWS)])
    for r in range(NPAD // 16):
        denom_v[pl.ds(r * 16, 16)] = zero16
    plsc.subcore_barrier()

    pltpu.sync_copy(wa_hbm, wav)
    wa = [wav[pl.ds(16 * d, 16)] for d in range(8)]
    ebase = wid * EW

    def chunk(i, carry):
        base = pl.multiple_of(ebase + i * C, 8)
        pltpu.sync_copy(dst_hbm.at[pl.ds(base, C)], dvec)
        pltpu.sync_copy(src_hbm.at[pl.ds(base, C)], svec)
        d1 = pltpu.async_copy(qt_hbm.at[dvec], qbuf, sem1)
        d2 = pltpu.async_copy(kt_hbm.at[svec], abuf, sem2)
        d1.wait()
        d2.wait()
        mask0 = lax.iota(jnp.int32, 16) == 0
        for row in range(C):
            acc = jnp.zeros((16,), _f32)
            for d in range(8):
                qv = qbuf[row, pl.ds(16 * d, 16)]
                kv = abuf[row, pl.ds(16 * d, 16)]
                xi = jnp.maximum(qv + kv, 0.0)
                abuf[row, pl.ds(16 * d, 16)] = xi
                acc = acc + xi * wa[d]
            logit = jnp.sum(acc, axis=0)
            exb = jnp.exp(jnp.full((16,), logit, _f32))
            exmat[row, :] = exb
            dstb = plsc.load_gather(
                dvec, [jnp.full((16,), row, jnp.int32)])
            plsc.addupdate_scatter(denom_v, [dstb], exb, mask=mask0)
            for d in range(8):
                abuf[row, pl.ds(16 * d, 16)] = (
                    abuf[row, pl.ds(16 * d, 16)] * exb)
        pltpu.sync_copy(abuf, a_sp.at[dvec], add=True)
        pltpu.sync_copy(exmat, ex_hbm.at[pl.ds(base, C)])
        return carry

    lax.fori_loop(0, NCHUNK, chunk, 0)
    pltpu.sync_copy(denom_v, dall_hbm.at[pl.ds(wid * NPAD, NPAD)])
    plsc.subcore_barrier()
    pltpu.sync_copy(a_sp.at[pl.ds(row0, ROWS_PER_TILE)],
                    apart_hbm.at[c, pl.ds(row0, ROWS_PER_TILE)])


def _p1(dst, src, qt, kt, w_a):
    mesh = plsc.VectorSubcoreMesh(core_axis_name="c", subcore_axis_name="s")
    f = pl.kernel(
        _p1_body,
        out_type=[jax.ShapeDtypeStruct((E, 16), _f32),
                  jax.ShapeDtypeStruct((NC, NPAD, DIM), _f32),
                  jax.ShapeDtypeStruct((NW * NPAD,), _f32)],
        mesh=mesh,
        compiler_params=pltpu.CompilerParams(needs_layout_passes=False),
        scratch_types=[
            pltpu.VMEM_SHARED((NPAD, DIM), _f32),
            pltpu.VMEM((C,), jnp.int32),
            pltpu.VMEM((C,), jnp.int32),
            pltpu.VMEM((C, DIM), _f32),
            pltpu.VMEM((C, DIM), _f32),
            pltpu.VMEM((C, 16), _f32),
            pltpu.VMEM((NPAD,), _f32),
            pltpu.VMEM((DIM,), _f32),
            pltpu.SemaphoreType.DMA,
            pltpu.SemaphoreType.DMA,
        ],
    )
    return f(dst, src, qt, kt, w_a)


def _p2_body(src_hbm, dst_hbm, ex_hbm, v_hbm,
             bpart_hbm,
             b_sp, svec, dvec, vbuf, bbuf, exmat, sem1):
    c = lax.axis_index("c")
    s = lax.axis_index("s")
    wid = s * NC + c

    zero16 = jnp.zeros((16,), _f32)
    for r in range(ZROWS):
        for col in range(DIM // 16):
            bbuf[r, pl.ds(col * 16, 16)] = zero16
    row0 = s * ROWS_PER_TILE
    for b in range(ROWS_PER_TILE // ZROWS):
        pltpu.sync_copy(bbuf.at[pl.ds(0, ZROWS)],
                        b_sp.at[pl.ds(row0 + b * ZROWS, ZROWS)])
    plsc.subcore_barrier()

    ebase = wid * EW

    def chunk(i, carry):
        base = pl.multiple_of(ebase + i * C, 8)
        pltpu.sync_copy(src_hbm.at[pl.ds(base, C)], svec)
        pltpu.sync_copy(dst_hbm.at[pl.ds(base, C)], dvec)
        pltpu.sync_copy(ex_hbm.at[pl.ds(base, C)], exmat)
        pltpu.async_copy(v_hbm.at[svec], vbuf, sem1).wait()
        for row in range(C):
            exb = exmat[row, :]
            for d in range(8):
                bbuf[row, pl.ds(16 * d, 16)] = (
                    vbuf[row, pl.ds(16 * d, 16)] * exb)
        pltpu.sync_copy(bbuf, b_sp.at[dvec], add=True)
        return carry

    lax.fori_loop(0, NCHUNK, chunk, 0)
    plsc.subcore_barrier()
    pltpu.sync_copy(b_sp.at[pl.ds(row0, ROWS_PER_TILE)],
                    bpart_hbm.at[c, pl.ds(row0, ROWS_PER_TILE)])


def _p2(src, dst, ex, v):
    mesh = plsc.VectorSubcoreMesh(core_axis_name="c", subcore_axis_name="s")
    f = pl.kernel(
        _p2_body,
        out_type=jax.ShapeDtypeStruct((NC, NPAD, DIM), _f32),
        mesh=mesh,
        compiler_params=pltpu.CompilerParams(needs_layout_passes=False),
        scratch_types=[
            pltpu.VMEM_SHARED((NPAD, DIM), _f32),
            pltpu.VMEM((C,), jnp.int32),
            pltpu.VMEM((C,), jnp.int32),
            pltpu.VMEM((C, DIM), _f32),
            pltpu.VMEM((C, DIM), _f32),
            pltpu.VMEM((C, 16), _f32),
            pltpu.SemaphoreType.DMA,
        ],
    )
    return f(src, dst, ex, v)


def _tc2_body(x_ref, ap_ref, bp_ref, dcol_ref, we_ref, wg_ref, bg_ref,
              g2_ref, b2_ref, w1_ref, b1_ref, w2_ref, b2f_ref,
              y2_ref, gsum_ref):
    i = pl.program_id(0)
    xb = x_ref[...]
    A = ap_ref[0] + ap_ref[1]
    b = bp_ref[0] + bp_ref[1]
    denom = jnp.sum(dcol_ref[...], axis=-1, keepdims=True)
    dn = (((1,), (1,)), ((), ()))
    z = (b + lax.dot_general(A, we_ref[...], dn, preferred_element_type=_f32)
         ) / (denom + 1e-9)
    deg = xb[:, DIM - 4:DIM - 3]
    gate = jax.nn.sigmoid(deg * wg_ref[...] + bg_ref[...])
    z = z * (1.0 + gate)
    y = xb + z
    m = jnp.mean(y, axis=-1, keepdims=True)
    var = jnp.mean((y - m) ** 2, axis=-1, keepdims=True)
    h2 = (y - m) * lax.rsqrt(var + 1e-5) * g2_ref[...] + b2_ref[...]
    pre = (lax.dot_general(h2, w1_ref[...], dn, preferred_element_type=_f32)
           + b1_ref[...])
    f1 = 0.5 * pre * (1.0 + lax.erf(pre * (2.0 ** -0.5)))
    ffn = lax.dot_general(f1, w2_ref[...], dn,
                          preferred_element_type=_f32) + b2f_ref[...]
    y2 = y + ffn
    y2_ref[...] = y2

    @pl.when(i == 0)
    def _():
        gsum_ref[...] = jnp.zeros_like(gsum_ref)

    gsum_ref[...] += jnp.sum(y2, axis=0, keepdims=True)


def _tc2(x, apart, bpart, dcols, WE, w_g, b_g, ln2_g, ln2_b,
         ffn_w1, ffn_b1, ffn_w2, ffn_b2):
    BR = 1000
    grid = N // BR
    full2 = lambda shape: pl.BlockSpec(shape, lambda i: (0, 0))
    vec = lambda n: pl.BlockSpec((n,), lambda i: (0,))
    blk = pl.BlockSpec((BR, DIM), lambda i: (i, 0))
    return pl.pallas_call(
        _tc2_body,
        grid=(grid,),
        in_specs=[blk,
                  pl.BlockSpec((NC, BR, DIM), lambda i: (0, i, 0)),
                  pl.BlockSpec((NC, BR, DIM), lambda i: (0, i, 0)),
                  pl.BlockSpec((BR, NW), lambda i: (i, 0)),
                  full2((DIM, DIM)), vec(DIM), vec(DIM), vec(DIM), vec(DIM),
                  full2((HID, DIM)), vec(HID), full2((DIM, HID)), vec(DIM)],
        out_specs=[blk, pl.BlockSpec((1, DIM), lambda i: (0, 0))],
        out_shape=[jax.ShapeDtypeStruct((N, DIM), _f32),
                   jax.ShapeDtypeStruct((1, DIM), _f32)],
    )(x, apart, bpart, dcols, WE, w_g, b_g, ln2_g, ln2_b,
      ffn_w1, ffn_b1, ffn_w2, ffn_b2)


def _tc3_body(y2_ref, gsum_ref, w1_ref, b1_ref, w2_ref, b2_ref, out_ref):
    g = gsum_ref[...] * (1.0 / N)
    dn = (((1,), (1,)), ((), ()))
    t = jnp.maximum(
        lax.dot_general(g, w1_ref[...], dn, preferred_element_type=_f32)
        + b1_ref[...], 0.0)
    sg = jax.nn.sigmoid(
        lax.dot_general(t, w2_ref[...], dn, preferred_element_type=_f32)
        + b2_ref[...])
    out_ref[...] = y2_ref[...] * sg


def _tc3(y2, gsum, se_w1, se_b1, se_w2, se_b2):
    BR = 2000
    grid = N // BR
    full2 = lambda shape: pl.BlockSpec(shape, lambda i: (0, 0))
    vec = lambda n: pl.BlockSpec((n,), lambda i: (0,))
    blk = pl.BlockSpec((BR, DIM), lambda i: (i, 0))
    return pl.pallas_call(
        _tc3_body,
        grid=(grid,),
        in_specs=[blk, pl.BlockSpec((1, DIM), lambda i: (0, 0)),
                  full2((R, DIM)), vec(R), full2((DIM, R)), vec(DIM)],
        out_specs=blk,
        out_shape=jax.ShapeDtypeStruct((N, DIM), _f32),
    )(y2, gsum, se_w1, se_b1, se_w2, se_b2)


def kernel(x, edge_index, WQ, WK, WV, WS, WE, w_a, w_g, b_g,
           ln1_g, ln1_b, ln2_g, ln2_b,
           ffn_w1, ffn_b1, ffn_w2, ffn_b2,
           se_w1, se_b1, se_w2, se_b2):
    src = edge_index[0]
    dst = edge_index[1]
    qt, kt, v = _tc1(x, WQ, WK, WV, WS, ln1_g, ln1_b)
    ex, apart, dall = _p1(dst, src, qt, kt, w_a)
    bpart = _p2(src, dst, ex, v)
    dcols = dall.reshape(NW, NPAD).T
    y2, gsum = _tc2(x, apart, bpart, dcols, WE, w_g, b_g, ln2_g, ln2_b,
                    ffn_w1, ffn_b1, ffn_w2, ffn_b2)
    return _tc3(y2, gsum, se_w1, se_b1, se_w2, se_b2)
